# baseline (speedup 1.0000x reference)
import functools

import jax
import jax.numpy as jnp
from jax import lax
from jax.experimental import pallas as pl
from jax.experimental.pallas import tpu as pltpu
from jax.experimental.pallas import tpu_sc as plsc

_BATCH, _HIST, _DIM = 4096, 50, 32
_NC, _NS = 2, 16
_NW = _NC * _NS
_BPW = _BATCH // _NW
_IPW = _BPW * _HIST
_G = 128
_NG = _IPW // _G


def _sc_lookup(table, idx_w, rep, zeros):
    mesh = plsc.VectorSubcoreMesh(core_axis_name="c", subcore_axis_name="s")

    @functools.partial(
        pl.kernel,
        out_type=jax.ShapeDtypeStruct((_BATCH, _DIM), jnp.float32),
        mesh=mesh,
        compiler_params=pltpu.CompilerParams(use_tc_tiling_on_sc=False),
        scratch_types=[
            pltpu.VMEM((_NG, _G), jnp.int32),
            pltpu.VMEM((_NG, _G), jnp.int32),
            pltpu.VMEM((_G, _DIM), jnp.float32),
            pltpu.VMEM_SHARED((_NS * _BPW, _DIM), jnp.float32),
            pltpu.SemaphoreType.DMA,
        ],
    )
    def k(table_hbm, idx_hbm, rep_hbm, zeros_hbm, out_hbm,
          idx_v, rep_v, rows_v, acc_sh, sem):
        c = lax.axis_index("c")
        s = lax.axis_index("s")
        wid = s * _NC + c
        pltpu.sync_copy(idx_hbm.at[wid], idx_v)
        pltpu.sync_copy(rep_hbm.at[s], rep_v)
        pltpu.sync_copy(zeros_hbm, acc_sh.at[pl.ds(s * _BPW, _BPW)])

        def body(j, carry):
            pltpu.async_copy(table_hbm.at[idx_v.at[j]], rows_v, sem).wait()
            pltpu.sync_copy(rows_v, acc_sh.at[rep_v.at[j]], add=True)
            return carry

        lax.fori_loop(0, _NG, body, 0)
        pltpu.sync_copy(acc_sh.at[pl.ds(s * _BPW, _BPW)],
                        out_hbm.at[pl.ds(wid * _BPW, _BPW)])

    return k(table, idx_w, rep, zeros)


def kernel(idx, table):
    idx_w = idx.reshape(_NW, _NG, _G)
    rep = (jnp.arange(_IPW, dtype=jnp.int32) // _HIST).reshape(1, _NG, _G)
    rep = rep + (jnp.arange(_NS, dtype=jnp.int32) * _BPW)[:, None, None]
    zeros = jnp.zeros((_BPW, _DIM), jnp.float32)
    out = _sc_lookup(table, idx_w, rep, zeros)
    return out[:, None, :]


# speedup vs baseline: 1.0555x; 1.0555x over previous
import functools

import jax
import jax.numpy as jnp
from jax import lax
from jax.experimental import pallas as pl
from jax.experimental.pallas import tpu as pltpu
from jax.experimental.pallas import tpu_sc as plsc

_BATCH, _HIST, _DIM = 4096, 50, 32
_NC, _NS = 2, 16
_NW = _NC * _NS
_BPW = _BATCH // _NW
_IPW = _BPW * _HIST
_G = 128
_NG = _IPW // _G
_K = 5
_NGRP = _NG // _K


def _sc_lookup(table, idx_w, rep, zeros):
    mesh = plsc.VectorSubcoreMesh(core_axis_name="c", subcore_axis_name="s")

    @functools.partial(
        pl.kernel,
        out_type=jax.ShapeDtypeStruct((_BATCH, _DIM), jnp.float32),
        mesh=mesh,
        compiler_params=pltpu.CompilerParams(use_tc_tiling_on_sc=False),
        scratch_types=[
            pltpu.VMEM((_NG, _G), jnp.int32),
            pltpu.VMEM((_NG, _G), jnp.int32),
            pltpu.VMEM((2, _K, _G, _DIM), jnp.float32),
            pltpu.VMEM_SHARED((_NS * _BPW, _DIM), jnp.float32),
            pltpu.SemaphoreType.DMA,
            pltpu.SemaphoreType.DMA,
        ],
    )
    def k(table_hbm, idx_hbm, rep_hbm, zeros_hbm, out_hbm,
          idx_v, rep_v, rows_v, acc_sh, gsem, ssem):
        c = lax.axis_index("c")
        s = lax.axis_index("s")
        wid = s * _NC + c
        pltpu.sync_copy(idx_hbm.at[wid], idx_v)
        pltpu.sync_copy(rep_hbm.at[s], rep_v)
        pltpu.sync_copy(zeros_hbm, acc_sh.at[pl.ds(s * _BPW, _BPW)])

        def fire_gathers(g, dst_set):
            return [pltpu.async_copy(table_hbm.at[idx_v.at[g * _K + b]],
                                     rows_v.at[dst_set, b], gsem)
                    for b in range(_K)]

        gd = fire_gathers(0, 0)
        sd = []
        for g in range(_NGRP):
            cur = g % 2
            for d in gd:
                d.wait()
            for d in sd:
                d.wait()
            if g + 1 < _NGRP:
                gd = fire_gathers(g + 1, 1 - cur)
            sd = [pltpu.async_copy(rows_v.at[cur, b],
                                   acc_sh.at[rep_v.at[g * _K + b]],
                                   ssem, add=True)
                  for b in range(_K)]
        for d in sd:
            d.wait()
        pltpu.sync_copy(acc_sh.at[pl.ds(s * _BPW, _BPW)],
                        out_hbm.at[pl.ds(wid * _BPW, _BPW)])

    return k(table, idx_w, rep, zeros)


def kernel(idx, table):
    idx_w = idx.reshape(_NW, _NG, _G)
    rep = (jnp.arange(_IPW, dtype=jnp.int32) // _HIST).reshape(1, _NG, _G)
    rep = rep + (jnp.arange(_NS, dtype=jnp.int32) * _BPW)[:, None, None]
    zeros = jnp.zeros((_BPW, _DIM), jnp.float32)
    out = _sc_lookup(table, idx_w, rep, zeros)
    return out[:, None, :]
